# Initial kernel scaffold; baseline (speedup 1.0000x reference)
import jax
import jax.numpy as jnp
from jax.experimental import pallas as pl


def kernel(coord0, feat0, coord1, feat1, coord2, feat2, target_coord, Wp0, bp0, Wp1, bp1, Wp2, bp2, Wf1, bf1, Wf2, bf2, Wq1, bq1, Wq2, bq2):
    raise NotImplementedError("write your pallas kernel here")


# speedup vs baseline: 2.4306x; 2.4306x over previous
import functools
import math

import jax
import jax.numpy as jnp
from jax import lax
from jax.experimental import pallas as pl

_HI = lax.Precision.HIGHEST


def _dot(a, b):
    return jnp.dot(a, b, precision=_HI, preferred_element_type=jnp.float32)


def _interp_body(tgt_ref, srcT_ref, feat_ref, wp_ref, bp_ref, out_ref, *, ns, k):
    tt = tgt_ref.shape[0]
    d = jnp.zeros((tt, ns), jnp.float32)
    for c in range(3):
        diff = tgt_ref[:, c:c + 1] - srcT_ref[c:c + 1, :]
        d = d + diff * diff
    iota = lax.broadcasted_iota(jnp.int32, (tt, ns), 1)
    wmat = jnp.zeros((tt, ns), jnp.float32)
    wsum = jnp.zeros((tt, 1), jnp.float32)
    dcur = d
    for _ in range(k):
        m = jnp.min(dcur, axis=1, keepdims=True)
        ismin = dcur == m
        idx = jnp.min(jnp.where(ismin, iota, ns), axis=1, keepdims=True)
        sel = iota == idx
        w = 1.0 / (m + 1e-8)
        wmat = wmat + jnp.where(sel, w, 0.0)
        wsum = wsum + w
        dcur = jnp.where(sel, jnp.float32(1e30), dcur)
    wmat = wmat / wsum
    a = _dot(wmat, feat_ref[...])
    out_ref[...] = _dot(a, wp_ref[...]) + bp_ref[...]


def _interp_project(tgt, srcT, feat, wp, bp2d, tt):
    nt = tgt.shape[0]
    ns, cs = feat.shape
    c_out = wp.shape[1]
    return pl.pallas_call(
        functools.partial(_interp_body, ns=ns, k=3),
        grid=(nt // tt,),
        in_specs=[
            pl.BlockSpec((tt, 3), lambda i: (i, 0)),
            pl.BlockSpec((3, ns), lambda i: (0, 0)),
            pl.BlockSpec((ns, cs), lambda i: (0, 0)),
            pl.BlockSpec((cs, c_out), lambda i: (0, 0)),
            pl.BlockSpec((1, c_out), lambda i: (0, 0)),
        ],
        out_specs=pl.BlockSpec((tt, c_out), lambda i: (i, 0)),
        out_shape=jax.ShapeDtypeStruct((nt, c_out), jnp.float32),
    )(tgt, srcT, feat, wp, bp2d)


def _fuse_body(a0_ref, a1_ref, a2_ref, w1_ref, b1_ref, w2_ref, b2_ref,
               out_ref, *, c):
    h = (_dot(a0_ref[...], w1_ref[0:c, :])
         + _dot(a1_ref[...], w1_ref[c:2 * c, :])
         + _dot(a2_ref[...], w1_ref[2 * c:3 * c, :])
         + b1_ref[...])
    h = 0.5 * h * (1.0 + lax.erf(h * jnp.float32(1.0 / math.sqrt(2.0))))
    out_ref[...] = _dot(h, w2_ref[...]) + b2_ref[...]


def _fuse(a0, a1, a2, wf1, bf1_2d, wf2, bf2_2d, tt):
    nt, c = a0.shape
    return pl.pallas_call(
        functools.partial(_fuse_body, c=c),
        grid=(nt // tt,),
        in_specs=[
            pl.BlockSpec((tt, c), lambda i: (i, 0)),
            pl.BlockSpec((tt, c), lambda i: (i, 0)),
            pl.BlockSpec((tt, c), lambda i: (i, 0)),
            pl.BlockSpec((3 * c, c), lambda i: (0, 0)),
            pl.BlockSpec((1, c), lambda i: (0, 0)),
            pl.BlockSpec((c, c), lambda i: (0, 0)),
            pl.BlockSpec((1, c), lambda i: (0, 0)),
        ],
        out_specs=pl.BlockSpec((tt, c), lambda i: (i, 0)),
        out_shape=jax.ShapeDtypeStruct((nt, c), jnp.float32),
    )(a0, a1, a2, wf1, bf1_2d, wf2, bf2_2d)


def _cgen_body(out_ref, *, n, tt):
    i0 = pl.program_id(0) * tt
    ii = lax.broadcasted_iota(jnp.int32, (tt, n), 0) + i0
    jj = lax.broadcasted_iota(jnp.int32, (tt, n), 1)
    m = (ii * jj) % n
    out_ref[...] = jnp.cos(m.astype(jnp.float32) * jnp.float32(2.0 * math.pi / n))


def _cgen(n, tt):
    return pl.pallas_call(
        functools.partial(_cgen_body, n=n, tt=tt),
        grid=(n // tt,),
        out_specs=pl.BlockSpec((tt, n), lambda i: (i, 0)),
        out_shape=jax.ShapeDtypeStruct((n, n), jnp.float32),
    )()


def _dft_mlp_body(c_ref, x_ref, w1_ref, b1_ref, w2_ref, b2_ref, out_ref):
    y = _dot(c_ref[...], x_ref[...])
    h = jnp.maximum(_dot(y, w1_ref[...]) + b1_ref[...], 0.0)
    out_ref[...] = _dot(h, w2_ref[...]) + b2_ref[...]


def _dft_mlp(cmat, x, wq1, bq1_2d, wq2, bq2_2d, tt):
    n, c = x.shape
    return pl.pallas_call(
        _dft_mlp_body,
        grid=(n // tt,),
        in_specs=[
            pl.BlockSpec((tt, n), lambda i: (i, 0)),
            pl.BlockSpec((n, c), lambda i: (0, 0)),
            pl.BlockSpec((c, c), lambda i: (0, 0)),
            pl.BlockSpec((1, c), lambda i: (0, 0)),
            pl.BlockSpec((c, c), lambda i: (0, 0)),
            pl.BlockSpec((1, c), lambda i: (0, 0)),
        ],
        out_specs=pl.BlockSpec((tt, c), lambda i: (i, 0)),
        out_shape=jax.ShapeDtypeStruct((n, c), jnp.float32),
    )(cmat, x, wq1, bq1_2d, wq2, bq2_2d)


def _dft_gate_body(c_ref, e_ref, f_ref, out_ref, *, n):
    xr = _dot(c_ref[...], e_ref[...]) * jnp.float32(1.0 / n)
    out_ref[...] = f_ref[...] * (1.0 + jax.nn.sigmoid(xr))


def _dft_gate(cmat, enh, fused, tt):
    n, c = enh.shape
    return pl.pallas_call(
        functools.partial(_dft_gate_body, n=n),
        grid=(n // tt,),
        in_specs=[
            pl.BlockSpec((tt, n), lambda i: (i, 0)),
            pl.BlockSpec((n, c), lambda i: (0, 0)),
            pl.BlockSpec((tt, c), lambda i: (i, 0)),
        ],
        out_specs=pl.BlockSpec((tt, c), lambda i: (i, 0)),
        out_shape=jax.ShapeDtypeStruct((n, c), jnp.float32),
    )(cmat, enh, fused)


def kernel(coord0, feat0, coord1, feat1, coord2, feat2, target_coord,
           Wp0, bp0, Wp1, bp1, Wp2, bp2, Wf1, bf1, Wf2, bf2,
           Wq1, bq1, Wq2, bq2):
    nt = target_coord.shape[0]
    tt_i = min(128, nt)
    tt_d = min(256, nt)

    aligned = []
    for coord, feat, wp, bp in ((coord0, feat0, Wp0, bp0),
                                (coord1, feat1, Wp1, bp1),
                                (coord2, feat2, Wp2, bp2)):
        aligned.append(_interp_project(target_coord, coord.T, feat, wp,
                                       bp.reshape(1, -1), tt_i))

    fused = _fuse(aligned[0], aligned[1], aligned[2],
                  Wf1, bf1.reshape(1, -1), Wf2, bf2.reshape(1, -1), tt_d)

    cmat = _cgen(nt, tt_d)
    enh = _dft_mlp(cmat, fused, Wq1, bq1.reshape(1, -1),
                   Wq2, bq2.reshape(1, -1), tt_d)
    return _dft_gate(cmat, enh, fused, tt_d)


# speedup vs baseline: 4.6908x; 1.9299x over previous
import functools
import math

import jax
import jax.numpy as jnp
from jax import lax
from jax.experimental import pallas as pl

_HI = lax.Precision.HIGHEST


def _dot(a, b):
    return jnp.dot(a, b, precision=_HI, preferred_element_type=jnp.float32)


def _interp_body(tgt_ref, srcT_ref, feat_ref, wp_ref, bp_ref, out_ref, *, ns, k):
    tt = tgt_ref.shape[0]
    d = jnp.zeros((tt, ns), jnp.float32)
    for c in range(3):
        diff = tgt_ref[:, c:c + 1] - srcT_ref[c:c + 1, :]
        d = d + diff * diff
    iota = lax.broadcasted_iota(jnp.int32, (tt, ns), 1)
    wmat = jnp.zeros((tt, ns), jnp.float32)
    wsum = jnp.zeros((tt, 1), jnp.float32)
    dcur = d
    for _ in range(k):
        m = jnp.min(dcur, axis=1, keepdims=True)
        ismin = dcur == m
        idx = jnp.min(jnp.where(ismin, iota, ns), axis=1, keepdims=True)
        sel = iota == idx
        w = 1.0 / (m + 1e-8)
        wmat = wmat + jnp.where(sel, w, 0.0)
        wsum = wsum + w
        dcur = jnp.where(sel, jnp.float32(1e30), dcur)
    wmat = wmat / wsum
    a = _dot(wmat, feat_ref[...])
    out_ref[...] = _dot(a, wp_ref[...]) + bp_ref[...]


def _interp_project(tgt, srcT, feat, wp, bp2d, tt):
    nt = tgt.shape[0]
    ns, cs = feat.shape
    c_out = wp.shape[1]
    return pl.pallas_call(
        functools.partial(_interp_body, ns=ns, k=3),
        grid=(nt // tt,),
        in_specs=[
            pl.BlockSpec((tt, 3), lambda i: (i, 0)),
            pl.BlockSpec((3, ns), lambda i: (0, 0)),
            pl.BlockSpec((ns, cs), lambda i: (0, 0)),
            pl.BlockSpec((cs, c_out), lambda i: (0, 0)),
            pl.BlockSpec((1, c_out), lambda i: (0, 0)),
        ],
        out_specs=pl.BlockSpec((tt, c_out), lambda i: (i, 0)),
        out_shape=jax.ShapeDtypeStruct((nt, c_out), jnp.float32),
    )(tgt, srcT, feat, wp, bp2d)


def _fuse_body(a0_ref, a1_ref, a2_ref, w1_ref, b1_ref, w2_ref, b2_ref,
               out_ref, *, c):
    h = (_dot(a0_ref[...], w1_ref[0:c, :])
         + _dot(a1_ref[...], w1_ref[c:2 * c, :])
         + _dot(a2_ref[...], w1_ref[2 * c:3 * c, :])
         + b1_ref[...])
    h = 0.5 * h * (1.0 + lax.erf(h * jnp.float32(1.0 / math.sqrt(2.0))))
    out_ref[...] = _dot(h, w2_ref[...]) + b2_ref[...]


def _fuse(a0, a1, a2, wf1, bf1_2d, wf2, bf2_2d, tt):
    nt, c = a0.shape
    return pl.pallas_call(
        functools.partial(_fuse_body, c=c),
        grid=(nt // tt,),
        in_specs=[
            pl.BlockSpec((tt, c), lambda i: (i, 0)),
            pl.BlockSpec((tt, c), lambda i: (i, 0)),
            pl.BlockSpec((tt, c), lambda i: (i, 0)),
            pl.BlockSpec((3 * c, c), lambda i: (0, 0)),
            pl.BlockSpec((1, c), lambda i: (0, 0)),
            pl.BlockSpec((c, c), lambda i: (0, 0)),
            pl.BlockSpec((1, c), lambda i: (0, 0)),
        ],
        out_specs=pl.BlockSpec((tt, c), lambda i: (i, 0)),
        out_shape=jax.ShapeDtypeStruct((nt, c), jnp.float32),
    )(a0, a1, a2, wf1, bf1_2d, wf2, bf2_2d)


import numpy as np


def _trig_consts(n, n1, n2):
    i1 = np.arange(n1)
    ph1 = 2.0 * np.pi * np.outer(i1, i1) / n1
    i2 = np.arange(n2)
    ph2 = 2.0 * np.pi * np.outer(i2, i2) / n2
    p = np.arange(n)
    php = 2.0 * np.pi * ((p // n2) * (p % n2) % n) / n
    f32 = lambda a: jnp.asarray(a, dtype=jnp.float32)
    return (f32(np.cos(ph1)), f32(np.sin(ph1)),
            f32(np.cos(ph2)), f32(np.sin(ph2)),
            f32(np.cos(php)[:, None]), f32(np.sin(php)[:, None]))


def _s1_body(c1_ref, s1_ref, x_ref, ar_ref, ai_ref):
    x = x_ref[...]
    ar_ref[...] = _dot(c1_ref[...], x)
    ai_ref[...] = -_dot(s1_ref[...], x)


def _s1(c1, s1, x2, ch):
    n1, m = x2.shape
    outs = (jax.ShapeDtypeStruct((n1, m), jnp.float32),) * 2
    return pl.pallas_call(
        _s1_body,
        grid=(m // ch,),
        in_specs=[
            pl.BlockSpec((n1, n1), lambda j: (0, 0)),
            pl.BlockSpec((n1, n1), lambda j: (0, 0)),
            pl.BlockSpec((n1, ch), lambda j: (0, j)),
        ],
        out_specs=[pl.BlockSpec((n1, ch), lambda j: (0, j))] * 2,
        out_shape=outs,
    )(c1, s1, x2)


def _s2_body(c2_ref, s2_ref, ar_ref, ai_ref, tc_ref, ts_ref,
             w1_ref, b1_ref, w2_ref, b2_ref, out_ref):
    ar, ai = ar_ref[...], ai_ref[...]
    tc, ts = tc_ref[...], ts_ref[...]
    br = tc * ar + ts * ai
    bi = tc * ai - ts * ar
    xr = _dot(c2_ref[...], br) + _dot(s2_ref[...], bi)
    h = jnp.maximum(_dot(xr, w1_ref[...]) + b1_ref[...], 0.0)
    out_ref[...] = _dot(h, w2_ref[...]) + b2_ref[...]


def _s2(c2, s2, ar, ai, tc, ts, wq1, bq1_2d, wq2, bq2_2d, n1, n2):
    n, c = ar.shape
    return pl.pallas_call(
        _s2_body,
        grid=(n1,),
        in_specs=[
            pl.BlockSpec((n2, n2), lambda k: (0, 0)),
            pl.BlockSpec((n2, n2), lambda k: (0, 0)),
            pl.BlockSpec((n2, c), lambda k: (k, 0)),
            pl.BlockSpec((n2, c), lambda k: (k, 0)),
            pl.BlockSpec((n2, 1), lambda k: (k, 0)),
            pl.BlockSpec((n2, 1), lambda k: (k, 0)),
            pl.BlockSpec((c, c), lambda k: (0, 0)),
            pl.BlockSpec((1, c), lambda k: (0, 0)),
            pl.BlockSpec((c, c), lambda k: (0, 0)),
            pl.BlockSpec((1, c), lambda k: (0, 0)),
        ],
        out_specs=pl.BlockSpec((n2, c), lambda k: (k, 0)),
        out_shape=jax.ShapeDtypeStruct((n, c), jnp.float32),
    )(c2, s2, ar, ai, tc, ts, wq1, bq1_2d, wq2, bq2_2d)


def _s3_body(c2_ref, s2_ref, e_ref, tc_ref, ts_ref, hr_ref, hi_ref):
    e = e_ref[...]
    gr = _dot(c2_ref[...], e)
    gi = -_dot(s2_ref[...], e)
    tc, ts = tc_ref[...], ts_ref[...]
    hr_ref[...] = tc * gr + ts * gi
    hi_ref[...] = tc * gi - ts * gr


def _s3(c2, s2, e, tc, ts, n1, n2):
    n, c = e.shape
    outs = (jax.ShapeDtypeStruct((n, c), jnp.float32),) * 2
    return pl.pallas_call(
        _s3_body,
        grid=(n1,),
        in_specs=[
            pl.BlockSpec((n2, n2), lambda k: (0, 0)),
            pl.BlockSpec((n2, n2), lambda k: (0, 0)),
            pl.BlockSpec((n2, c), lambda k: (k, 0)),
            pl.BlockSpec((n2, 1), lambda k: (k, 0)),
            pl.BlockSpec((n2, 1), lambda k: (k, 0)),
        ],
        out_specs=[pl.BlockSpec((n2, c), lambda k: (k, 0))] * 2,
        out_shape=outs,
    )(c2, s2, e, tc, ts)


def _s4_body(c1_ref, s1_ref, hr_ref, hi_ref, f_ref, out_ref, *, n):
    r = (_dot(c1_ref[...], hr_ref[...]) + _dot(s1_ref[...], hi_ref[...]))
    r = r * jnp.float32(1.0 / n)
    out_ref[...] = f_ref[...] * (1.0 + jax.nn.sigmoid(r))


def _s4(c1, s1, hr2, hi2, fused2, ch, n):
    n1, m = hr2.shape
    return pl.pallas_call(
        functools.partial(_s4_body, n=n),
        grid=(m // ch,),
        in_specs=[
            pl.BlockSpec((n1, n1), lambda j: (0, 0)),
            pl.BlockSpec((n1, n1), lambda j: (0, 0)),
            pl.BlockSpec((n1, ch), lambda j: (0, j)),
            pl.BlockSpec((n1, ch), lambda j: (0, j)),
            pl.BlockSpec((n1, ch), lambda j: (0, j)),
        ],
        out_specs=pl.BlockSpec((n1, ch), lambda j: (0, j)),
        out_shape=jax.ShapeDtypeStruct((n1, m), jnp.float32),
    )(c1, s1, hr2, hi2, fused2)


def kernel(coord0, feat0, coord1, feat1, coord2, feat2, target_coord,
           Wp0, bp0, Wp1, bp1, Wp2, bp2, Wf1, bf1, Wf2, bf2,
           Wq1, bq1, Wq2, bq2):
    nt = target_coord.shape[0]
    tt_i = min(128, nt)
    tt_d = min(256, nt)

    aligned = []
    for coord, feat, wp, bp in ((coord0, feat0, Wp0, bp0),
                                (coord1, feat1, Wp1, bp1),
                                (coord2, feat2, Wp2, bp2)):
        aligned.append(_interp_project(target_coord, coord.T, feat, wp,
                                       bp.reshape(1, -1), tt_i))

    fused = _fuse(aligned[0], aligned[1], aligned[2],
                  Wf1, bf1.reshape(1, -1), Wf2, bf2.reshape(1, -1), tt_d)

    c = fused.shape[1]
    if nt % (64 * 128) == 0:
        n1, n2 = 64, nt // 64
    else:
        n1 = max(d for d in range(1, int(math.isqrt(nt)) + 1) if nt % d == 0)
        n2 = nt // n1
    c1, s1, c2, s2, tw_c, tw_s = _trig_consts(nt, n1, n2)
    ch = min(2048, n2 * c)

    x2 = fused.reshape(n1, n2 * c)
    ar2, ai2 = _s1(c1, s1, x2, ch)
    enh = _s2(c2, s2, ar2.reshape(nt, c), ai2.reshape(nt, c), tw_c, tw_s,
              Wq1, bq1.reshape(1, -1), Wq2, bq2.reshape(1, -1), n1, n2)
    hr, hi = _s3(c2, s2, enh, tw_c, tw_s, n1, n2)
    out2 = _s4(c1, s1, hr.reshape(n1, n2 * c), hi.reshape(n1, n2 * c),
               x2, ch, nt)
    return out2.reshape(nt, c)


# speedup vs baseline: 7.5699x; 1.6138x over previous
import functools
import math

import jax
import jax.numpy as jnp
from jax import lax
from jax.experimental import pallas as pl

_HI = lax.Precision.HIGHEST


def _dot(a, b, prec=_HI):
    return jnp.dot(a, b, precision=prec, preferred_element_type=jnp.float32)


def _interp_body(tgt_ref, srcT_ref, feat_ref, wp_ref, bp_ref, out_ref, *, ns, k):
    tt = tgt_ref.shape[0]
    d = jnp.zeros((tt, ns), jnp.float32)
    for c in range(3):
        diff = tgt_ref[:, c:c + 1] - srcT_ref[c:c + 1, :]
        d = d + diff * diff
    iota = lax.broadcasted_iota(jnp.int32, (tt, ns), 1)
    wmat = jnp.zeros((tt, ns), jnp.float32)
    wsum = jnp.zeros((tt, 1), jnp.float32)
    dcur = d
    for _ in range(k):
        m = jnp.min(dcur, axis=1, keepdims=True)
        ismin = dcur == m
        idx = jnp.min(jnp.where(ismin, iota, ns), axis=1, keepdims=True)
        sel = iota == idx
        w = 1.0 / (m + 1e-8)
        wmat = wmat + jnp.where(sel, w, 0.0)
        wsum = wsum + w
        dcur = jnp.where(sel, jnp.float32(1e30), dcur)
    wmat = wmat / wsum
    a = _dot(wmat, feat_ref[...], lax.Precision.DEFAULT)
    out_ref[...] = _dot(a, wp_ref[...], lax.Precision.DEFAULT) + bp_ref[...]


def _interp_project(tgt, srcT, feat, wp, bp2d, tt):
    nt = tgt.shape[0]
    ns, cs = feat.shape
    c_out = wp.shape[1]
    return pl.pallas_call(
        functools.partial(_interp_body, ns=ns, k=3),
        grid=(nt // tt,),
        in_specs=[
            pl.BlockSpec((tt, 3), lambda i: (i, 0)),
            pl.BlockSpec((3, ns), lambda i: (0, 0)),
            pl.BlockSpec((ns, cs), lambda i: (0, 0)),
            pl.BlockSpec((cs, c_out), lambda i: (0, 0)),
            pl.BlockSpec((1, c_out), lambda i: (0, 0)),
        ],
        out_specs=pl.BlockSpec((tt, c_out), lambda i: (i, 0)),
        out_shape=jax.ShapeDtypeStruct((nt, c_out), jnp.float32),
    )(tgt, srcT, feat, wp, bp2d)


def _fuse_body(a0_ref, a1_ref, a2_ref, w1_ref, b1_ref, w2_ref, b2_ref,
               out_ref, *, c):
    h = (_dot(a0_ref[...], w1_ref[0:c, :], lax.Precision.DEFAULT)
         + _dot(a1_ref[...], w1_ref[c:2 * c, :], lax.Precision.DEFAULT)
         + _dot(a2_ref[...], w1_ref[2 * c:3 * c, :], lax.Precision.DEFAULT)
         + b1_ref[...])
    h = 0.5 * h * (1.0 + lax.erf(h * jnp.float32(1.0 / math.sqrt(2.0))))
    out_ref[...] = _dot(h, w2_ref[...], lax.Precision.DEFAULT) + b2_ref[...]


def _fuse(a0, a1, a2, wf1, bf1_2d, wf2, bf2_2d, tt):
    nt, c = a0.shape
    return pl.pallas_call(
        functools.partial(_fuse_body, c=c),
        grid=(nt // tt,),
        in_specs=[
            pl.BlockSpec((tt, c), lambda i: (i, 0)),
            pl.BlockSpec((tt, c), lambda i: (i, 0)),
            pl.BlockSpec((tt, c), lambda i: (i, 0)),
            pl.BlockSpec((3 * c, c), lambda i: (0, 0)),
            pl.BlockSpec((1, c), lambda i: (0, 0)),
            pl.BlockSpec((c, c), lambda i: (0, 0)),
            pl.BlockSpec((1, c), lambda i: (0, 0)),
        ],
        out_specs=pl.BlockSpec((tt, c), lambda i: (i, 0)),
        out_shape=jax.ShapeDtypeStruct((nt, c), jnp.float32),
    )(a0, a1, a2, wf1, bf1_2d, wf2, bf2_2d)


import numpy as np


def _trig_consts(n, n1, n2):
    i1 = np.arange(n1)
    ph1 = 2.0 * np.pi * np.outer(i1, i1) / n1
    i2 = np.arange(n2)
    ph2 = 2.0 * np.pi * np.outer(i2, i2) / n2
    p = np.arange(n)
    php = 2.0 * np.pi * ((p // n2) * (p % n2) % n) / n
    f32 = lambda a: jnp.asarray(a, dtype=jnp.float32)
    return (f32(np.cos(ph1)), f32(np.sin(ph1)),
            f32(np.cos(ph2)), f32(np.sin(ph2)),
            f32(np.cos(php)[:, None]), f32(np.sin(php)[:, None]))


def _s1_body(c1_ref, s1_ref, x_ref, ar_ref, ai_ref):
    x = x_ref[...]
    ar_ref[...] = _dot(c1_ref[...], x)
    ai_ref[...] = -_dot(s1_ref[...], x)


def _s1(c1, s1, x2, ch):
    n1, m = x2.shape
    outs = (jax.ShapeDtypeStruct((n1, m), jnp.float32),) * 2
    return pl.pallas_call(
        _s1_body,
        grid=(m // ch,),
        in_specs=[
            pl.BlockSpec((n1, n1), lambda j: (0, 0)),
            pl.BlockSpec((n1, n1), lambda j: (0, 0)),
            pl.BlockSpec((n1, ch), lambda j: (0, j)),
        ],
        out_specs=[pl.BlockSpec((n1, ch), lambda j: (0, j))] * 2,
        out_shape=outs,
    )(c1, s1, x2)


def _s2_body(c2_ref, s2_ref, ar_ref, ai_ref, tc_ref, ts_ref,
             w1_ref, b1_ref, w2_ref, b2_ref, out_ref):
    ar, ai = ar_ref[...], ai_ref[...]
    tc, ts = tc_ref[...], ts_ref[...]
    br = tc * ar + ts * ai
    bi = tc * ai - ts * ar
    xr = _dot(c2_ref[...], br) + _dot(s2_ref[...], bi)
    h = jnp.maximum(_dot(xr, w1_ref[...]) + b1_ref[...], 0.0)
    out_ref[...] = _dot(h, w2_ref[...]) + b2_ref[...]


def _s2(c2, s2, ar, ai, tc, ts, wq1, bq1_2d, wq2, bq2_2d, n1, n2):
    n, c = ar.shape
    return pl.pallas_call(
        _s2_body,
        grid=(n1,),
        in_specs=[
            pl.BlockSpec((n2, n2), lambda k: (0, 0)),
            pl.BlockSpec((n2, n2), lambda k: (0, 0)),
            pl.BlockSpec((n2, c), lambda k: (k, 0)),
            pl.BlockSpec((n2, c), lambda k: (k, 0)),
            pl.BlockSpec((n2, 1), lambda k: (k, 0)),
            pl.BlockSpec((n2, 1), lambda k: (k, 0)),
            pl.BlockSpec((c, c), lambda k: (0, 0)),
            pl.BlockSpec((1, c), lambda k: (0, 0)),
            pl.BlockSpec((c, c), lambda k: (0, 0)),
            pl.BlockSpec((1, c), lambda k: (0, 0)),
        ],
        out_specs=pl.BlockSpec((n2, c), lambda k: (k, 0)),
        out_shape=jax.ShapeDtypeStruct((n, c), jnp.float32),
    )(c2, s2, ar, ai, tc, ts, wq1, bq1_2d, wq2, bq2_2d)


def _s3_body(c2_ref, s2_ref, e_ref, tc_ref, ts_ref, hr_ref, hi_ref):
    e = e_ref[...]
    gr = _dot(c2_ref[...], e)
    gi = -_dot(s2_ref[...], e)
    tc, ts = tc_ref[...], ts_ref[...]
    hr_ref[...] = tc * gr + ts * gi
    hi_ref[...] = tc * gi - ts * gr


def _s3(c2, s2, e, tc, ts, n1, n2):
    n, c = e.shape
    outs = (jax.ShapeDtypeStruct((n, c), jnp.float32),) * 2
    return pl.pallas_call(
        _s3_body,
        grid=(n1,),
        in_specs=[
            pl.BlockSpec((n2, n2), lambda k: (0, 0)),
            pl.BlockSpec((n2, n2), lambda k: (0, 0)),
            pl.BlockSpec((n2, c), lambda k: (k, 0)),
            pl.BlockSpec((n2, 1), lambda k: (k, 0)),
            pl.BlockSpec((n2, 1), lambda k: (k, 0)),
        ],
        out_specs=[pl.BlockSpec((n2, c), lambda k: (k, 0))] * 2,
        out_shape=outs,
    )(c2, s2, e, tc, ts)


def _s4_body(c1_ref, s1_ref, hr_ref, hi_ref, f_ref, out_ref, *, n):
    r = (_dot(c1_ref[...], hr_ref[...]) + _dot(s1_ref[...], hi_ref[...]))
    r = r * jnp.float32(1.0 / n)
    out_ref[...] = f_ref[...] * (1.0 + jax.nn.sigmoid(r))


def _s4(c1, s1, hr2, hi2, fused2, ch, n):
    n1, m = hr2.shape
    return pl.pallas_call(
        functools.partial(_s4_body, n=n),
        grid=(m // ch,),
        in_specs=[
            pl.BlockSpec((n1, n1), lambda j: (0, 0)),
            pl.BlockSpec((n1, n1), lambda j: (0, 0)),
            pl.BlockSpec((n1, ch), lambda j: (0, j)),
            pl.BlockSpec((n1, ch), lambda j: (0, j)),
            pl.BlockSpec((n1, ch), lambda j: (0, j)),
        ],
        out_specs=pl.BlockSpec((n1, ch), lambda j: (0, j)),
        out_shape=jax.ShapeDtypeStruct((n1, m), jnp.float32),
    )(c1, s1, hr2, hi2, fused2)


def kernel(coord0, feat0, coord1, feat1, coord2, feat2, target_coord,
           Wp0, bp0, Wp1, bp1, Wp2, bp2, Wf1, bf1, Wf2, bf2,
           Wq1, bq1, Wq2, bq2):
    nt = target_coord.shape[0]
    tt_i = min(128, nt)
    tt_d = min(256, nt)

    aligned = []
    for coord, feat, wp, bp in ((coord0, feat0, Wp0, bp0),
                                (coord1, feat1, Wp1, bp1),
                                (coord2, feat2, Wp2, bp2)):
        aligned.append(_interp_project(target_coord, coord.T, feat, wp,
                                       bp.reshape(1, -1), tt_i))

    fused = _fuse(aligned[0], aligned[1], aligned[2],
                  Wf1, bf1.reshape(1, -1), Wf2, bf2.reshape(1, -1), tt_d)

    c = fused.shape[1]
    if nt % (64 * 128) == 0:
        n1, n2 = 64, nt // 64
    else:
        n1 = max(d for d in range(1, int(math.isqrt(nt)) + 1) if nt % d == 0)
        n2 = nt // n1
    c1, s1, c2, s2, tw_c, tw_s = _trig_consts(nt, n1, n2)
    ch = min(2048, n2 * c)

    x2 = fused.reshape(n1, n2 * c)
    ar2, ai2 = _s1(c1, s1, x2, ch)
    enh = _s2(c2, s2, ar2.reshape(nt, c), ai2.reshape(nt, c), tw_c, tw_s,
              Wq1, bq1.reshape(1, -1), Wq2, bq2.reshape(1, -1), n1, n2)
    hr, hi = _s3(c2, s2, enh, tw_c, tw_s, n1, n2)
    out2 = _s4(c1, s1, hr.reshape(n1, n2 * c), hi.reshape(n1, n2 * c),
               x2, ch, nt)
    return out2.reshape(nt, c)


# speedup vs baseline: 8.2132x; 1.0850x over previous
import functools
import math

import jax
import jax.numpy as jnp
from jax import lax
from jax.experimental import pallas as pl

_HI = lax.Precision.HIGHEST


def _dot(a, b, prec=_HI):
    return jnp.dot(a, b, precision=prec, preferred_element_type=jnp.float32)


def _interp_block(tgt_ref, srcT_ref, feat_ref, wp_ref, bp_ref, *, ns, k=3):
    tt = tgt_ref.shape[0]
    d = jnp.zeros((tt, ns), jnp.float32)
    for c in range(3):
        diff = tgt_ref[:, c:c + 1] - srcT_ref[c:c + 1, :]
        d = d + diff * diff
    iota = lax.broadcasted_iota(jnp.int32, (tt, ns), 1)
    wmat = jnp.zeros((tt, ns), jnp.float32)
    wsum = jnp.zeros((tt, 1), jnp.float32)
    dcur = d
    for _ in range(k):
        m = jnp.min(dcur, axis=1, keepdims=True)
        ismin = dcur == m
        idx = jnp.min(jnp.where(ismin, iota, ns), axis=1, keepdims=True)
        sel = iota == idx
        w = 1.0 / (m + 1e-8)
        wmat = wmat + jnp.where(sel, w, 0.0)
        wsum = wsum + w
        dcur = jnp.where(sel, jnp.float32(1e30), dcur)
    wmat = wmat / wsum
    a = _dot(wmat, feat_ref[...], lax.Precision.DEFAULT)
    return _dot(a, wp_ref[...], lax.Precision.DEFAULT) + bp_ref[...]


def _front_body(tgt_ref,
                s0_ref, f0_ref, wp0_ref, bp0_ref,
                s1_ref, f1_ref, wp1_ref, bp1_ref,
                s2_ref, f2_ref, wp2_ref, bp2_ref,
                w1_ref, b1_ref, w2_ref, b2_ref,
                out_ref, *, ns0, ns1, ns2, c):
    a0 = _interp_block(tgt_ref, s0_ref, f0_ref, wp0_ref, bp0_ref, ns=ns0)
    a1 = _interp_block(tgt_ref, s1_ref, f1_ref, wp1_ref, bp1_ref, ns=ns1)
    a2 = _interp_block(tgt_ref, s2_ref, f2_ref, wp2_ref, bp2_ref, ns=ns2)
    h = (_dot(a0, w1_ref[0:c, :], lax.Precision.DEFAULT)
         + _dot(a1, w1_ref[c:2 * c, :], lax.Precision.DEFAULT)
         + _dot(a2, w1_ref[2 * c:3 * c, :], lax.Precision.DEFAULT)
         + b1_ref[...])
    h = 0.5 * h * (1.0 + lax.erf(h * jnp.float32(1.0 / math.sqrt(2.0))))
    out_ref[...] = _dot(h, w2_ref[...], lax.Precision.DEFAULT) + b2_ref[...]


def _front(tgt, srcs, feats, wps, bps, wf1, bf1_2d, wf2, bf2_2d, tt):
    nt = tgt.shape[0]
    c = wps[0].shape[1]
    ns = [f.shape[0] for f in feats]
    cs = [f.shape[1] for f in feats]
    full = lambda shape: pl.BlockSpec(shape, lambda i: (0, 0))
    in_specs = [pl.BlockSpec((tt, 3), lambda i: (i, 0))]
    operands = [tgt]
    for l in range(3):
        in_specs += [full((3, ns[l])), full((ns[l], cs[l])),
                     full((cs[l], c)), full((1, c))]
        operands += [srcs[l], feats[l], wps[l], bps[l]]
    in_specs += [full((3 * c, c)), full((1, c)), full((c, c)), full((1, c))]
    operands += [wf1, bf1_2d, wf2, bf2_2d]
    return pl.pallas_call(
        functools.partial(_front_body, ns0=ns[0], ns1=ns[1], ns2=ns[2], c=c),
        grid=(nt // tt,),
        in_specs=in_specs,
        out_specs=pl.BlockSpec((tt, c), lambda i: (i, 0)),
        out_shape=jax.ShapeDtypeStruct((nt, c), jnp.float32),
    )(*operands)


import numpy as np


def _trig_consts(n, n1, n2):
    i1 = np.arange(n1)
    ph1 = 2.0 * np.pi * np.outer(i1, i1) / n1
    i2 = np.arange(n2)
    ph2 = 2.0 * np.pi * np.outer(i2, i2) / n2
    p = np.arange(n)
    php = 2.0 * np.pi * ((p // n2) * (p % n2) % n) / n
    f32 = lambda a: jnp.asarray(a, dtype=jnp.float32)
    return (f32(np.cos(ph1)), f32(np.sin(ph1)),
            f32(np.cos(ph2)), f32(np.sin(ph2)),
            f32(np.cos(php)[:, None]), f32(np.sin(php)[:, None]))


def _s1_body(c1_ref, s1_ref, x_ref, ar_ref, ai_ref):
    x = x_ref[...]
    ar_ref[...] = _dot(c1_ref[...], x)
    ai_ref[...] = -_dot(s1_ref[...], x)


def _s1(c1, s1, x2, ch):
    n1, m = x2.shape
    outs = (jax.ShapeDtypeStruct((n1, m), jnp.float32),) * 2
    return pl.pallas_call(
        _s1_body,
        grid=(m // ch,),
        in_specs=[
            pl.BlockSpec((n1, n1), lambda j: (0, 0)),
            pl.BlockSpec((n1, n1), lambda j: (0, 0)),
            pl.BlockSpec((n1, ch), lambda j: (0, j)),
        ],
        out_specs=[pl.BlockSpec((n1, ch), lambda j: (0, j))] * 2,
        out_shape=outs,
    )(c1, s1, x2)


def _s2_body(c2_ref, s2_ref, ar_ref, ai_ref, tc_ref, ts_ref,
             w1_ref, b1_ref, w2_ref, b2_ref, out_ref):
    ar, ai = ar_ref[...], ai_ref[...]
    tc, ts = tc_ref[...], ts_ref[...]
    br = tc * ar + ts * ai
    bi = tc * ai - ts * ar
    xr = _dot(c2_ref[...], br) + _dot(s2_ref[...], bi)
    h = jnp.maximum(_dot(xr, w1_ref[...]) + b1_ref[...], 0.0)
    out_ref[...] = _dot(h, w2_ref[...]) + b2_ref[...]


def _s2(c2, s2, ar, ai, tc, ts, wq1, bq1_2d, wq2, bq2_2d, n1, n2):
    n, c = ar.shape
    return pl.pallas_call(
        _s2_body,
        grid=(n1,),
        in_specs=[
            pl.BlockSpec((n2, n2), lambda k: (0, 0)),
            pl.BlockSpec((n2, n2), lambda k: (0, 0)),
            pl.BlockSpec((n2, c), lambda k: (k, 0)),
            pl.BlockSpec((n2, c), lambda k: (k, 0)),
            pl.BlockSpec((n2, 1), lambda k: (k, 0)),
            pl.BlockSpec((n2, 1), lambda k: (k, 0)),
            pl.BlockSpec((c, c), lambda k: (0, 0)),
            pl.BlockSpec((1, c), lambda k: (0, 0)),
            pl.BlockSpec((c, c), lambda k: (0, 0)),
            pl.BlockSpec((1, c), lambda k: (0, 0)),
        ],
        out_specs=pl.BlockSpec((n2, c), lambda k: (k, 0)),
        out_shape=jax.ShapeDtypeStruct((n, c), jnp.float32),
    )(c2, s2, ar, ai, tc, ts, wq1, bq1_2d, wq2, bq2_2d)


def _s3_body(c2_ref, s2_ref, e_ref, tc_ref, ts_ref, hr_ref, hi_ref):
    e = e_ref[...]
    gr = _dot(c2_ref[...], e)
    gi = -_dot(s2_ref[...], e)
    tc, ts = tc_ref[...], ts_ref[...]
    hr_ref[...] = tc * gr + ts * gi
    hi_ref[...] = tc * gi - ts * gr


def _s3(c2, s2, e, tc, ts, n1, n2):
    n, c = e.shape
    outs = (jax.ShapeDtypeStruct((n, c), jnp.float32),) * 2
    return pl.pallas_call(
        _s3_body,
        grid=(n1,),
        in_specs=[
            pl.BlockSpec((n2, n2), lambda k: (0, 0)),
            pl.BlockSpec((n2, n2), lambda k: (0, 0)),
            pl.BlockSpec((n2, c), lambda k: (k, 0)),
            pl.BlockSpec((n2, 1), lambda k: (k, 0)),
            pl.BlockSpec((n2, 1), lambda k: (k, 0)),
        ],
        out_specs=[pl.BlockSpec((n2, c), lambda k: (k, 0))] * 2,
        out_shape=outs,
    )(c2, s2, e, tc, ts)


def _s4_body(c1_ref, s1_ref, hr_ref, hi_ref, f_ref, out_ref, *, n):
    r = (_dot(c1_ref[...], hr_ref[...]) + _dot(s1_ref[...], hi_ref[...]))
    r = r * jnp.float32(1.0 / n)
    out_ref[...] = f_ref[...] * (1.0 + jax.nn.sigmoid(r))


def _s4(c1, s1, hr2, hi2, fused2, ch, n):
    n1, m = hr2.shape
    return pl.pallas_call(
        functools.partial(_s4_body, n=n),
        grid=(m // ch,),
        in_specs=[
            pl.BlockSpec((n1, n1), lambda j: (0, 0)),
            pl.BlockSpec((n1, n1), lambda j: (0, 0)),
            pl.BlockSpec((n1, ch), lambda j: (0, j)),
            pl.BlockSpec((n1, ch), lambda j: (0, j)),
            pl.BlockSpec((n1, ch), lambda j: (0, j)),
        ],
        out_specs=pl.BlockSpec((n1, ch), lambda j: (0, j)),
        out_shape=jax.ShapeDtypeStruct((n1, m), jnp.float32),
    )(c1, s1, hr2, hi2, fused2)


def kernel(coord0, feat0, coord1, feat1, coord2, feat2, target_coord,
           Wp0, bp0, Wp1, bp1, Wp2, bp2, Wf1, bf1, Wf2, bf2,
           Wq1, bq1, Wq2, bq2):
    nt = target_coord.shape[0]
    tt_i = min(128, nt)
    tt_d = min(256, nt)

    fused = _front(target_coord,
                   (coord0.T, coord1.T, coord2.T),
                   (feat0, feat1, feat2),
                   (Wp0, Wp1, Wp2),
                   (bp0.reshape(1, -1), bp1.reshape(1, -1), bp2.reshape(1, -1)),
                   Wf1, bf1.reshape(1, -1), Wf2, bf2.reshape(1, -1), tt_i)

    c = fused.shape[1]
    if nt % (64 * 128) == 0:
        n1, n2 = 64, nt // 64
    else:
        n1 = max(d for d in range(1, int(math.isqrt(nt)) + 1) if nt % d == 0)
        n2 = nt // n1
    c1, s1, c2, s2, tw_c, tw_s = _trig_consts(nt, n1, n2)
    ch = min(2048, n2 * c)

    x2 = fused.reshape(n1, n2 * c)
    ar2, ai2 = _s1(c1, s1, x2, ch)
    enh = _s2(c2, s2, ar2.reshape(nt, c), ai2.reshape(nt, c), tw_c, tw_s,
              Wq1, bq1.reshape(1, -1), Wq2, bq2.reshape(1, -1), n1, n2)
    hr, hi = _s3(c2, s2, enh, tw_c, tw_s, n1, n2)
    out2 = _s4(c1, s1, hr.reshape(n1, n2 * c), hi.reshape(n1, n2 * c),
               x2, ch, nt)
    return out2.reshape(nt, c)


# speedup vs baseline: 8.7334x; 1.0633x over previous
import functools
import math

import jax
import jax.numpy as jnp
from jax import lax
from jax.experimental import pallas as pl

_HI = lax.Precision.HIGHEST


def _dot(a, b, prec=_HI):
    return jnp.dot(a, b, precision=prec, preferred_element_type=jnp.float32)


def _interp_block(tgt_ref, srcT_ref, feat_ref, wp_ref, bp_ref, *, ns, k=3):
    tt = tgt_ref.shape[0]
    d = jnp.zeros((tt, ns), jnp.float32)
    for c in range(3):
        diff = tgt_ref[:, c:c + 1] - srcT_ref[c:c + 1, :]
        d = d + diff * diff
    iota = lax.broadcasted_iota(jnp.int32, (tt, ns), 1)
    wmat = jnp.zeros((tt, ns), jnp.float32)
    wsum = jnp.zeros((tt, 1), jnp.float32)
    dcur = d
    for _ in range(k):
        m = jnp.min(dcur, axis=1, keepdims=True)
        ismin = dcur == m
        idx = jnp.min(jnp.where(ismin, iota, ns), axis=1, keepdims=True)
        sel = iota == idx
        w = 1.0 / (m + 1e-8)
        wmat = wmat + jnp.where(sel, w, 0.0)
        wsum = wsum + w
        dcur = jnp.where(sel, jnp.float32(1e30), dcur)
    wmat = wmat / wsum
    a = _dot(wmat, feat_ref[...], lax.Precision.DEFAULT)
    return _dot(a, wp_ref[...], lax.Precision.DEFAULT) + bp_ref[...]


def _front_body(tgt_ref,
                s0_ref, f0_ref, wp0_ref, bp0_ref,
                s1_ref, f1_ref, wp1_ref, bp1_ref,
                s2_ref, f2_ref, wp2_ref, bp2_ref,
                w1_ref, b1_ref, w2_ref, b2_ref,
                out_ref, *, ns0, ns1, ns2, c):
    a0 = _interp_block(tgt_ref, s0_ref, f0_ref, wp0_ref, bp0_ref, ns=ns0)
    a1 = _interp_block(tgt_ref, s1_ref, f1_ref, wp1_ref, bp1_ref, ns=ns1)
    a2 = _interp_block(tgt_ref, s2_ref, f2_ref, wp2_ref, bp2_ref, ns=ns2)
    h = (_dot(a0, w1_ref[0:c, :], lax.Precision.DEFAULT)
         + _dot(a1, w1_ref[c:2 * c, :], lax.Precision.DEFAULT)
         + _dot(a2, w1_ref[2 * c:3 * c, :], lax.Precision.DEFAULT)
         + b1_ref[...])
    h = 0.5 * h * (1.0 + lax.erf(h * jnp.float32(1.0 / math.sqrt(2.0))))
    out_ref[...] = _dot(h, w2_ref[...], lax.Precision.DEFAULT) + b2_ref[...]


def _front(tgt, srcs, feats, wps, bps, wf1, bf1_2d, wf2, bf2_2d, tt):
    nt = tgt.shape[0]
    c = wps[0].shape[1]
    ns = [f.shape[0] for f in feats]
    cs = [f.shape[1] for f in feats]
    full = lambda shape: pl.BlockSpec(shape, lambda i: (0, 0))
    in_specs = [pl.BlockSpec((tt, 3), lambda i: (i, 0))]
    operands = [tgt]
    for l in range(3):
        in_specs += [full((3, ns[l])), full((ns[l], cs[l])),
                     full((cs[l], c)), full((1, c))]
        operands += [srcs[l], feats[l], wps[l], bps[l]]
    in_specs += [full((3 * c, c)), full((1, c)), full((c, c)), full((1, c))]
    operands += [wf1, bf1_2d, wf2, bf2_2d]
    return pl.pallas_call(
        functools.partial(_front_body, ns0=ns[0], ns1=ns[1], ns2=ns[2], c=c),
        grid=(nt // tt,),
        in_specs=in_specs,
        out_specs=pl.BlockSpec((tt, c), lambda i: (i, 0)),
        out_shape=jax.ShapeDtypeStruct((nt, c), jnp.float32),
    )(*operands)


import numpy as np


def _trig_consts(n, n1, n2):
    i1 = np.arange(n1)
    ph1 = 2.0 * np.pi * np.outer(i1, i1) / n1
    i2 = np.arange(n2)
    ph2 = 2.0 * np.pi * np.outer(i2, i2) / n2
    p = np.arange(n)
    php = 2.0 * np.pi * ((p // n2) * (p % n2) % n) / n
    f32 = lambda a: jnp.asarray(a, dtype=jnp.float32)
    return (f32(np.cos(ph1)), f32(np.sin(ph1)),
            f32(np.cos(ph2)), f32(np.sin(ph2)),
            f32(np.cos(php)[:, None]), f32(np.sin(php)[:, None]))


def _s1_body(c1_ref, s1_ref, x_ref, ar_ref, ai_ref):
    x = x_ref[...]
    ar_ref[...] = _dot(c1_ref[...], x)
    ai_ref[...] = -_dot(s1_ref[...], x)


def _s1(c1, s1, x2, ch):
    n1, m = x2.shape
    outs = (jax.ShapeDtypeStruct((n1, m), jnp.float32),) * 2
    return pl.pallas_call(
        _s1_body,
        grid=(m // ch,),
        in_specs=[
            pl.BlockSpec((n1, n1), lambda j: (0, 0)),
            pl.BlockSpec((n1, n1), lambda j: (0, 0)),
            pl.BlockSpec((n1, ch), lambda j: (0, j)),
        ],
        out_specs=[pl.BlockSpec((n1, ch), lambda j: (0, j))] * 2,
        out_shape=outs,
    )(c1, s1, x2)


def _s2_body(c2_ref, s2_ref, ar_ref, ai_ref, tc_ref, ts_ref,
             w1_ref, b1_ref, w2_ref, b2_ref, out_ref):
    ar, ai = ar_ref[...], ai_ref[...]
    tc, ts = tc_ref[...], ts_ref[...]
    br = tc * ar + ts * ai
    bi = tc * ai - ts * ar
    xr = _dot(c2_ref[...], br) + _dot(s2_ref[...], bi)
    h = jnp.maximum(_dot(xr, w1_ref[...], lax.Precision.DEFAULT)
                    + b1_ref[...], 0.0)
    out_ref[...] = _dot(h, w2_ref[...], lax.Precision.DEFAULT) + b2_ref[...]


def _s2(c2, s2, ar, ai, tc, ts, wq1, bq1_2d, wq2, bq2_2d, n1, n2):
    n, c = ar.shape
    return pl.pallas_call(
        _s2_body,
        grid=(n1,),
        in_specs=[
            pl.BlockSpec((n2, n2), lambda k: (0, 0)),
            pl.BlockSpec((n2, n2), lambda k: (0, 0)),
            pl.BlockSpec((n2, c), lambda k: (k, 0)),
            pl.BlockSpec((n2, c), lambda k: (k, 0)),
            pl.BlockSpec((n2, 1), lambda k: (k, 0)),
            pl.BlockSpec((n2, 1), lambda k: (k, 0)),
            pl.BlockSpec((c, c), lambda k: (0, 0)),
            pl.BlockSpec((1, c), lambda k: (0, 0)),
            pl.BlockSpec((c, c), lambda k: (0, 0)),
            pl.BlockSpec((1, c), lambda k: (0, 0)),
        ],
        out_specs=pl.BlockSpec((n2, c), lambda k: (k, 0)),
        out_shape=jax.ShapeDtypeStruct((n, c), jnp.float32),
    )(c2, s2, ar, ai, tc, ts, wq1, bq1_2d, wq2, bq2_2d)


def _s3_body(c2_ref, s2_ref, e_ref, tc_ref, ts_ref, hr_ref, hi_ref):
    e = e_ref[...]
    gr = _dot(c2_ref[...], e)
    gi = -_dot(s2_ref[...], e)
    tc, ts = tc_ref[...], ts_ref[...]
    hr_ref[...] = tc * gr + ts * gi
    hi_ref[...] = tc * gi - ts * gr


def _s3(c2, s2, e, tc, ts, n1, n2):
    n, c = e.shape
    outs = (jax.ShapeDtypeStruct((n, c), jnp.float32),) * 2
    return pl.pallas_call(
        _s3_body,
        grid=(n1,),
        in_specs=[
            pl.BlockSpec((n2, n2), lambda k: (0, 0)),
            pl.BlockSpec((n2, n2), lambda k: (0, 0)),
            pl.BlockSpec((n2, c), lambda k: (k, 0)),
            pl.BlockSpec((n2, 1), lambda k: (k, 0)),
            pl.BlockSpec((n2, 1), lambda k: (k, 0)),
        ],
        out_specs=[pl.BlockSpec((n2, c), lambda k: (k, 0))] * 2,
        out_shape=outs,
    )(c2, s2, e, tc, ts)


def _s4_body(c1_ref, s1_ref, hr_ref, hi_ref, f_ref, out_ref, *, n):
    r = (_dot(c1_ref[...], hr_ref[...]) + _dot(s1_ref[...], hi_ref[...]))
    r = r * jnp.float32(1.0 / n)
    out_ref[...] = f_ref[...] * (1.0 + jax.nn.sigmoid(r))


def _s4(c1, s1, hr2, hi2, fused2, ch, n):
    n1, m = hr2.shape
    return pl.pallas_call(
        functools.partial(_s4_body, n=n),
        grid=(m // ch,),
        in_specs=[
            pl.BlockSpec((n1, n1), lambda j: (0, 0)),
            pl.BlockSpec((n1, n1), lambda j: (0, 0)),
            pl.BlockSpec((n1, ch), lambda j: (0, j)),
            pl.BlockSpec((n1, ch), lambda j: (0, j)),
            pl.BlockSpec((n1, ch), lambda j: (0, j)),
        ],
        out_specs=pl.BlockSpec((n1, ch), lambda j: (0, j)),
        out_shape=jax.ShapeDtypeStruct((n1, m), jnp.float32),
    )(c1, s1, hr2, hi2, fused2)


def kernel(coord0, feat0, coord1, feat1, coord2, feat2, target_coord,
           Wp0, bp0, Wp1, bp1, Wp2, bp2, Wf1, bf1, Wf2, bf2,
           Wq1, bq1, Wq2, bq2):
    nt = target_coord.shape[0]
    tt_i = min(128, nt)
    tt_d = min(256, nt)

    fused = _front(target_coord,
                   (coord0.T, coord1.T, coord2.T),
                   (feat0, feat1, feat2),
                   (Wp0, Wp1, Wp2),
                   (bp0.reshape(1, -1), bp1.reshape(1, -1), bp2.reshape(1, -1)),
                   Wf1, bf1.reshape(1, -1), Wf2, bf2.reshape(1, -1), tt_i)

    c = fused.shape[1]
    if nt % (64 * 128) == 0:
        n1, n2 = 64, nt // 64
    else:
        n1 = max(d for d in range(1, int(math.isqrt(nt)) + 1) if nt % d == 0)
        n2 = nt // n1
    c1, s1, c2, s2, tw_c, tw_s = _trig_consts(nt, n1, n2)
    ch = min(2048, n2 * c)

    x2 = fused.reshape(n1, n2 * c)
    ar2, ai2 = _s1(c1, s1, x2, ch)
    enh = _s2(c2, s2, ar2.reshape(nt, c), ai2.reshape(nt, c), tw_c, tw_s,
              Wq1, bq1.reshape(1, -1), Wq2, bq2.reshape(1, -1), n1, n2)
    hr, hi = _s3(c2, s2, enh, tw_c, tw_s, n1, n2)
    out2 = _s4(c1, s1, hr.reshape(n1, n2 * c), hi.reshape(n1, n2 * c),
               x2, ch, nt)
    return out2.reshape(nt, c)


# speedup vs baseline: 10.2729x; 1.1763x over previous
import functools
import math

import jax
import jax.numpy as jnp
from jax import lax
from jax.experimental import pallas as pl

_HI = lax.Precision.HIGHEST


def _dot(a, b, prec=_HI):
    return jnp.dot(a, b, precision=prec, preferred_element_type=jnp.float32)


def _interp_block(tgt_ref, srcT_ref, feat_ref, wp_ref, bp_ref, *, ns, k=3):
    tt = tgt_ref.shape[0]
    d = jnp.zeros((tt, ns), jnp.float32)
    for c in range(3):
        diff = tgt_ref[:, c:c + 1] - srcT_ref[c:c + 1, :]
        d = d + diff * diff
    big = jnp.float32(1e30)
    mcur = d
    for _ in range(k - 1):
        m = jnp.min(mcur, axis=1, keepdims=True)
        mcur = jnp.where(mcur == m, big, mcur)
    m3 = jnp.min(mcur, axis=1, keepdims=True)
    wmat = jnp.where(d <= m3, 1.0 / (d + 1e-8), 0.0)
    wsum = jnp.sum(wmat, axis=1, keepdims=True)
    a = _dot(wmat, feat_ref[...], lax.Precision.DEFAULT)
    a = a * (1.0 / wsum)
    return _dot(a, wp_ref[...], lax.Precision.DEFAULT) + bp_ref[...]


def _front_body(tgt_ref,
                s0_ref, f0_ref, wp0_ref, bp0_ref,
                s1_ref, f1_ref, wp1_ref, bp1_ref,
                s2_ref, f2_ref, wp2_ref, bp2_ref,
                w1_ref, b1_ref, w2_ref, b2_ref,
                out_ref, *, ns0, ns1, ns2, c):
    a0 = _interp_block(tgt_ref, s0_ref, f0_ref, wp0_ref, bp0_ref, ns=ns0)
    a1 = _interp_block(tgt_ref, s1_ref, f1_ref, wp1_ref, bp1_ref, ns=ns1)
    a2 = _interp_block(tgt_ref, s2_ref, f2_ref, wp2_ref, bp2_ref, ns=ns2)
    h = (_dot(a0, w1_ref[0:c, :], lax.Precision.DEFAULT)
         + _dot(a1, w1_ref[c:2 * c, :], lax.Precision.DEFAULT)
         + _dot(a2, w1_ref[2 * c:3 * c, :], lax.Precision.DEFAULT)
         + b1_ref[...])
    h = 0.5 * h * (1.0 + lax.erf(h * jnp.float32(1.0 / math.sqrt(2.0))))
    out_ref[...] = _dot(h, w2_ref[...], lax.Precision.DEFAULT) + b2_ref[...]


def _front(tgt, srcs, feats, wps, bps, wf1, bf1_2d, wf2, bf2_2d, tt):
    nt = tgt.shape[0]
    c = wps[0].shape[1]
    ns = [f.shape[0] for f in feats]
    cs = [f.shape[1] for f in feats]
    full = lambda shape: pl.BlockSpec(shape, lambda i: (0, 0))
    in_specs = [pl.BlockSpec((tt, 3), lambda i: (i, 0))]
    operands = [tgt]
    for l in range(3):
        in_specs += [full((3, ns[l])), full((ns[l], cs[l])),
                     full((cs[l], c)), full((1, c))]
        operands += [srcs[l], feats[l], wps[l], bps[l]]
    in_specs += [full((3 * c, c)), full((1, c)), full((c, c)), full((1, c))]
    operands += [wf1, bf1_2d, wf2, bf2_2d]
    return pl.pallas_call(
        functools.partial(_front_body, ns0=ns[0], ns1=ns[1], ns2=ns[2], c=c),
        grid=(nt // tt,),
        in_specs=in_specs,
        out_specs=pl.BlockSpec((tt, c), lambda i: (i, 0)),
        out_shape=jax.ShapeDtypeStruct((nt, c), jnp.float32),
    )(*operands)


import numpy as np


def _trig_consts(n, n1, n2):
    i1 = np.arange(n1)
    ph1 = 2.0 * np.pi * np.outer(i1, i1) / n1
    i2 = np.arange(n2)
    ph2 = 2.0 * np.pi * np.outer(i2, i2) / n2
    p = np.arange(n)
    php = 2.0 * np.pi * ((p // n2) * (p % n2) % n) / n
    f32 = lambda a: jnp.asarray(a, dtype=jnp.float32)
    return (f32(np.cos(ph1)), f32(np.sin(ph1)),
            f32(np.cos(ph2)), f32(np.sin(ph2)),
            f32(np.cos(php)[:, None]), f32(np.sin(php)[:, None]))


def _s1_body(c1_ref, s1_ref, x_ref, ar_ref, ai_ref):
    x = x_ref[...]
    ar_ref[...] = _dot(c1_ref[...], x)
    ai_ref[...] = -_dot(s1_ref[...], x)


def _s1(c1, s1, x2, ch):
    n1, m = x2.shape
    outs = (jax.ShapeDtypeStruct((n1, m), jnp.float32),) * 2
    return pl.pallas_call(
        _s1_body,
        grid=(m // ch,),
        in_specs=[
            pl.BlockSpec((n1, n1), lambda j: (0, 0)),
            pl.BlockSpec((n1, n1), lambda j: (0, 0)),
            pl.BlockSpec((n1, ch), lambda j: (0, j)),
        ],
        out_specs=[pl.BlockSpec((n1, ch), lambda j: (0, j))] * 2,
        out_shape=outs,
    )(c1, s1, x2)


def _s2_body(c2_ref, s2_ref, ar_ref, ai_ref, tc_ref, ts_ref,
             w1_ref, b1_ref, w2_ref, b2_ref, out_ref):
    ar, ai = ar_ref[...], ai_ref[...]
    tc, ts = tc_ref[...], ts_ref[...]
    br = tc * ar + ts * ai
    bi = tc * ai - ts * ar
    xr = _dot(c2_ref[...], br) + _dot(s2_ref[...], bi)
    h = jnp.maximum(_dot(xr, w1_ref[...], lax.Precision.DEFAULT)
                    + b1_ref[...], 0.0)
    out_ref[...] = _dot(h, w2_ref[...], lax.Precision.DEFAULT) + b2_ref[...]


def _s2(c2, s2, ar, ai, tc, ts, wq1, bq1_2d, wq2, bq2_2d, n1, n2):
    n, c = ar.shape
    return pl.pallas_call(
        _s2_body,
        grid=(n1,),
        in_specs=[
            pl.BlockSpec((n2, n2), lambda k: (0, 0)),
            pl.BlockSpec((n2, n2), lambda k: (0, 0)),
            pl.BlockSpec((n2, c), lambda k: (k, 0)),
            pl.BlockSpec((n2, c), lambda k: (k, 0)),
            pl.BlockSpec((n2, 1), lambda k: (k, 0)),
            pl.BlockSpec((n2, 1), lambda k: (k, 0)),
            pl.BlockSpec((c, c), lambda k: (0, 0)),
            pl.BlockSpec((1, c), lambda k: (0, 0)),
            pl.BlockSpec((c, c), lambda k: (0, 0)),
            pl.BlockSpec((1, c), lambda k: (0, 0)),
        ],
        out_specs=pl.BlockSpec((n2, c), lambda k: (k, 0)),
        out_shape=jax.ShapeDtypeStruct((n, c), jnp.float32),
    )(c2, s2, ar, ai, tc, ts, wq1, bq1_2d, wq2, bq2_2d)


def _s3_body(c2_ref, s2_ref, e_ref, tc_ref, ts_ref, hr_ref, hi_ref):
    e = e_ref[...]
    gr = _dot(c2_ref[...], e)
    gi = -_dot(s2_ref[...], e)
    tc, ts = tc_ref[...], ts_ref[...]
    hr_ref[...] = tc * gr + ts * gi
    hi_ref[...] = tc * gi - ts * gr


def _s3(c2, s2, e, tc, ts, n1, n2):
    n, c = e.shape
    outs = (jax.ShapeDtypeStruct((n, c), jnp.float32),) * 2
    return pl.pallas_call(
        _s3_body,
        grid=(n1,),
        in_specs=[
            pl.BlockSpec((n2, n2), lambda k: (0, 0)),
            pl.BlockSpec((n2, n2), lambda k: (0, 0)),
            pl.BlockSpec((n2, c), lambda k: (k, 0)),
            pl.BlockSpec((n2, 1), lambda k: (k, 0)),
            pl.BlockSpec((n2, 1), lambda k: (k, 0)),
        ],
        out_specs=[pl.BlockSpec((n2, c), lambda k: (k, 0))] * 2,
        out_shape=outs,
    )(c2, s2, e, tc, ts)


def _s4_body(c1_ref, s1_ref, hr_ref, hi_ref, f_ref, out_ref, *, n):
    r = (_dot(c1_ref[...], hr_ref[...]) + _dot(s1_ref[...], hi_ref[...]))
    r = r * jnp.float32(1.0 / n)
    out_ref[...] = f_ref[...] * (1.0 + jax.nn.sigmoid(r))


def _s4(c1, s1, hr2, hi2, fused2, ch, n):
    n1, m = hr2.shape
    return pl.pallas_call(
        functools.partial(_s4_body, n=n),
        grid=(m // ch,),
        in_specs=[
            pl.BlockSpec((n1, n1), lambda j: (0, 0)),
            pl.BlockSpec((n1, n1), lambda j: (0, 0)),
            pl.BlockSpec((n1, ch), lambda j: (0, j)),
            pl.BlockSpec((n1, ch), lambda j: (0, j)),
            pl.BlockSpec((n1, ch), lambda j: (0, j)),
        ],
        out_specs=pl.BlockSpec((n1, ch), lambda j: (0, j)),
        out_shape=jax.ShapeDtypeStruct((n1, m), jnp.float32),
    )(c1, s1, hr2, hi2, fused2)


def kernel(coord0, feat0, coord1, feat1, coord2, feat2, target_coord,
           Wp0, bp0, Wp1, bp1, Wp2, bp2, Wf1, bf1, Wf2, bf2,
           Wq1, bq1, Wq2, bq2):
    nt = target_coord.shape[0]
    tt_i = min(128, nt)
    tt_d = min(256, nt)

    fused = _front(target_coord,
                   (coord0.T, coord1.T, coord2.T),
                   (feat0, feat1, feat2),
                   (Wp0, Wp1, Wp2),
                   (bp0.reshape(1, -1), bp1.reshape(1, -1), bp2.reshape(1, -1)),
                   Wf1, bf1.reshape(1, -1), Wf2, bf2.reshape(1, -1), tt_i)

    c = fused.shape[1]
    if nt % (64 * 128) == 0:
        n1, n2 = 64, nt // 64
    else:
        n1 = max(d for d in range(1, int(math.isqrt(nt)) + 1) if nt % d == 0)
        n2 = nt // n1
    c1, s1, c2, s2, tw_c, tw_s = _trig_consts(nt, n1, n2)
    ch = min(2048, n2 * c)

    x2 = fused.reshape(n1, n2 * c)
    ar2, ai2 = _s1(c1, s1, x2, ch)
    enh = _s2(c2, s2, ar2.reshape(nt, c), ai2.reshape(nt, c), tw_c, tw_s,
              Wq1, bq1.reshape(1, -1), Wq2, bq2.reshape(1, -1), n1, n2)
    hr, hi = _s3(c2, s2, enh, tw_c, tw_s, n1, n2)
    out2 = _s4(c1, s1, hr.reshape(n1, n2 * c), hi.reshape(n1, n2 * c),
               x2, ch, nt)
    return out2.reshape(nt, c)


# speedup vs baseline: 10.8477x; 1.0560x over previous
import functools
import math

import jax
import jax.numpy as jnp
from jax import lax
from jax.experimental import pallas as pl

_HI = lax.Precision.DEFAULT


def _dot(a, b, prec=_HI):
    return jnp.dot(a, b, precision=prec, preferred_element_type=jnp.float32)


def _interp_block(tgt_ref, srcT_ref, feat_ref, wp_ref, bp_ref, *, ns, k=3):
    tt = tgt_ref.shape[0]
    d = jnp.zeros((tt, ns), jnp.float32)
    for c in range(3):
        diff = tgt_ref[:, c:c + 1] - srcT_ref[c:c + 1, :]
        d = d + diff * diff
    big = jnp.float32(1e30)
    mcur = d
    for _ in range(k - 1):
        m = jnp.min(mcur, axis=1, keepdims=True)
        mcur = jnp.where(mcur == m, big, mcur)
    m3 = jnp.min(mcur, axis=1, keepdims=True)
    wmat = jnp.where(d <= m3, 1.0 / (d + 1e-8), 0.0)
    wsum = jnp.sum(wmat, axis=1, keepdims=True)
    a = _dot(wmat, feat_ref[...], lax.Precision.DEFAULT)
    a = a * (1.0 / wsum)
    return _dot(a, wp_ref[...], lax.Precision.DEFAULT) + bp_ref[...]


def _front_body(tgt_ref,
                s0_ref, f0_ref, wp0_ref, bp0_ref,
                s1_ref, f1_ref, wp1_ref, bp1_ref,
                s2_ref, f2_ref, wp2_ref, bp2_ref,
                w1_ref, b1_ref, w2_ref, b2_ref,
                out_ref, *, ns0, ns1, ns2, c):
    a0 = _interp_block(tgt_ref, s0_ref, f0_ref, wp0_ref, bp0_ref, ns=ns0)
    a1 = _interp_block(tgt_ref, s1_ref, f1_ref, wp1_ref, bp1_ref, ns=ns1)
    a2 = _interp_block(tgt_ref, s2_ref, f2_ref, wp2_ref, bp2_ref, ns=ns2)
    h = (_dot(a0, w1_ref[0:c, :], lax.Precision.DEFAULT)
         + _dot(a1, w1_ref[c:2 * c, :], lax.Precision.DEFAULT)
         + _dot(a2, w1_ref[2 * c:3 * c, :], lax.Precision.DEFAULT)
         + b1_ref[...])
    h = 0.5 * h * (1.0 + lax.erf(h * jnp.float32(1.0 / math.sqrt(2.0))))
    out_ref[...] = _dot(h, w2_ref[...], lax.Precision.DEFAULT) + b2_ref[...]


def _front(tgt, srcs, feats, wps, bps, wf1, bf1_2d, wf2, bf2_2d, tt):
    nt = tgt.shape[0]
    c = wps[0].shape[1]
    ns = [f.shape[0] for f in feats]
    cs = [f.shape[1] for f in feats]
    full = lambda shape: pl.BlockSpec(shape, lambda i: (0, 0))
    in_specs = [pl.BlockSpec((tt, 3), lambda i: (i, 0))]
    operands = [tgt]
    for l in range(3):
        in_specs += [full((3, ns[l])), full((ns[l], cs[l])),
                     full((cs[l], c)), full((1, c))]
        operands += [srcs[l], feats[l], wps[l], bps[l]]
    in_specs += [full((3 * c, c)), full((1, c)), full((c, c)), full((1, c))]
    operands += [wf1, bf1_2d, wf2, bf2_2d]
    return pl.pallas_call(
        functools.partial(_front_body, ns0=ns[0], ns1=ns[1], ns2=ns[2], c=c),
        grid=(nt // tt,),
        in_specs=in_specs,
        out_specs=pl.BlockSpec((tt, c), lambda i: (i, 0)),
        out_shape=jax.ShapeDtypeStruct((nt, c), jnp.float32),
    )(*operands)


import numpy as np


def _trig_consts(n, n1, n2):
    i1 = np.arange(n1)
    ph1 = 2.0 * np.pi * np.outer(i1, i1) / n1
    i2 = np.arange(n2)
    ph2 = 2.0 * np.pi * np.outer(i2, i2) / n2
    p = np.arange(n)
    php = 2.0 * np.pi * ((p // n2) * (p % n2) % n) / n
    f32 = lambda a: jnp.asarray(a, dtype=jnp.float32)
    return (f32(np.cos(ph1)), f32(np.sin(ph1)),
            f32(np.cos(ph2)), f32(np.sin(ph2)),
            f32(np.cos(php)[:, None]), f32(np.sin(php)[:, None]))


def _s1_body(c1_ref, s1_ref, x_ref, ar_ref, ai_ref):
    x = x_ref[...]
    ar_ref[...] = _dot(c1_ref[...], x)
    ai_ref[...] = -_dot(s1_ref[...], x)


def _s1(c1, s1, x2, ch):
    n1, m = x2.shape
    outs = (jax.ShapeDtypeStruct((n1, m), jnp.float32),) * 2
    return pl.pallas_call(
        _s1_body,
        grid=(m // ch,),
        in_specs=[
            pl.BlockSpec((n1, n1), lambda j: (0, 0)),
            pl.BlockSpec((n1, n1), lambda j: (0, 0)),
            pl.BlockSpec((n1, ch), lambda j: (0, j)),
        ],
        out_specs=[pl.BlockSpec((n1, ch), lambda j: (0, j))] * 2,
        out_shape=outs,
    )(c1, s1, x2)


def _s2_body(c2_ref, s2_ref, ar_ref, ai_ref, tc_ref, ts_ref,
             w1_ref, b1_ref, w2_ref, b2_ref, out_ref):
    ar, ai = ar_ref[...], ai_ref[...]
    tc, ts = tc_ref[...], ts_ref[...]
    br = tc * ar + ts * ai
    bi = tc * ai - ts * ar
    xr = _dot(c2_ref[...], br) + _dot(s2_ref[...], bi)
    h = jnp.maximum(_dot(xr, w1_ref[...], lax.Precision.DEFAULT)
                    + b1_ref[...], 0.0)
    out_ref[...] = _dot(h, w2_ref[...], lax.Precision.DEFAULT) + b2_ref[...]


def _s2(c2, s2, ar, ai, tc, ts, wq1, bq1_2d, wq2, bq2_2d, n1, n2):
    n, c = ar.shape
    return pl.pallas_call(
        _s2_body,
        grid=(n1,),
        in_specs=[
            pl.BlockSpec((n2, n2), lambda k: (0, 0)),
            pl.BlockSpec((n2, n2), lambda k: (0, 0)),
            pl.BlockSpec((n2, c), lambda k: (k, 0)),
            pl.BlockSpec((n2, c), lambda k: (k, 0)),
            pl.BlockSpec((n2, 1), lambda k: (k, 0)),
            pl.BlockSpec((n2, 1), lambda k: (k, 0)),
            pl.BlockSpec((c, c), lambda k: (0, 0)),
            pl.BlockSpec((1, c), lambda k: (0, 0)),
            pl.BlockSpec((c, c), lambda k: (0, 0)),
            pl.BlockSpec((1, c), lambda k: (0, 0)),
        ],
        out_specs=pl.BlockSpec((n2, c), lambda k: (k, 0)),
        out_shape=jax.ShapeDtypeStruct((n, c), jnp.float32),
    )(c2, s2, ar, ai, tc, ts, wq1, bq1_2d, wq2, bq2_2d)


def _s3_body(c2_ref, s2_ref, e_ref, tc_ref, ts_ref, hr_ref, hi_ref):
    e = e_ref[...]
    gr = _dot(c2_ref[...], e)
    gi = -_dot(s2_ref[...], e)
    tc, ts = tc_ref[...], ts_ref[...]
    hr_ref[...] = tc * gr + ts * gi
    hi_ref[...] = tc * gi - ts * gr


def _s3(c2, s2, e, tc, ts, n1, n2):
    n, c = e.shape
    outs = (jax.ShapeDtypeStruct((n, c), jnp.float32),) * 2
    return pl.pallas_call(
        _s3_body,
        grid=(n1,),
        in_specs=[
            pl.BlockSpec((n2, n2), lambda k: (0, 0)),
            pl.BlockSpec((n2, n2), lambda k: (0, 0)),
            pl.BlockSpec((n2, c), lambda k: (k, 0)),
            pl.BlockSpec((n2, 1), lambda k: (k, 0)),
            pl.BlockSpec((n2, 1), lambda k: (k, 0)),
        ],
        out_specs=[pl.BlockSpec((n2, c), lambda k: (k, 0))] * 2,
        out_shape=outs,
    )(c2, s2, e, tc, ts)


def _s4_body(c1_ref, s1_ref, hr_ref, hi_ref, f_ref, out_ref, *, n):
    r = (_dot(c1_ref[...], hr_ref[...]) + _dot(s1_ref[...], hi_ref[...]))
    r = r * jnp.float32(1.0 / n)
    out_ref[...] = f_ref[...] * (1.0 + jax.nn.sigmoid(r))


def _s4(c1, s1, hr2, hi2, fused2, ch, n):
    n1, m = hr2.shape
    return pl.pallas_call(
        functools.partial(_s4_body, n=n),
        grid=(m // ch,),
        in_specs=[
            pl.BlockSpec((n1, n1), lambda j: (0, 0)),
            pl.BlockSpec((n1, n1), lambda j: (0, 0)),
            pl.BlockSpec((n1, ch), lambda j: (0, j)),
            pl.BlockSpec((n1, ch), lambda j: (0, j)),
            pl.BlockSpec((n1, ch), lambda j: (0, j)),
        ],
        out_specs=pl.BlockSpec((n1, ch), lambda j: (0, j)),
        out_shape=jax.ShapeDtypeStruct((n1, m), jnp.float32),
    )(c1, s1, hr2, hi2, fused2)


def kernel(coord0, feat0, coord1, feat1, coord2, feat2, target_coord,
           Wp0, bp0, Wp1, bp1, Wp2, bp2, Wf1, bf1, Wf2, bf2,
           Wq1, bq1, Wq2, bq2):
    nt = target_coord.shape[0]
    tt_i = min(128, nt)
    tt_d = min(256, nt)

    fused = _front(target_coord,
                   (coord0.T, coord1.T, coord2.T),
                   (feat0, feat1, feat2),
                   (Wp0, Wp1, Wp2),
                   (bp0.reshape(1, -1), bp1.reshape(1, -1), bp2.reshape(1, -1)),
                   Wf1, bf1.reshape(1, -1), Wf2, bf2.reshape(1, -1), tt_i)

    c = fused.shape[1]
    if nt % (64 * 128) == 0:
        n1, n2 = 64, nt // 64
    else:
        n1 = max(d for d in range(1, int(math.isqrt(nt)) + 1) if nt % d == 0)
        n2 = nt // n1
    c1, s1, c2, s2, tw_c, tw_s = _trig_consts(nt, n1, n2)
    ch = min(2048, n2 * c)

    x2 = fused.reshape(n1, n2 * c)
    ar2, ai2 = _s1(c1, s1, x2, ch)
    enh = _s2(c2, s2, ar2.reshape(nt, c), ai2.reshape(nt, c), tw_c, tw_s,
              Wq1, bq1.reshape(1, -1), Wq2, bq2.reshape(1, -1), n1, n2)
    hr, hi = _s3(c2, s2, enh, tw_c, tw_s, n1, n2)
    out2 = _s4(c1, s1, hr.reshape(n1, n2 * c), hi.reshape(n1, n2 * c),
               x2, ch, nt)
    return out2.reshape(nt, c)


# speedup vs baseline: 11.7373x; 1.0820x over previous
import functools
import math

import jax
import jax.numpy as jnp
from jax import lax
from jax.experimental import pallas as pl

_HI = lax.Precision.DEFAULT


def _dot(a, b, prec=_HI):
    return jnp.dot(a, b, precision=prec, preferred_element_type=jnp.float32)


def _interp_block(tgt_ref, srcT_ref, feat_ref, wp_ref, bp_ref, *, ns, k=3):
    tt = tgt_ref.shape[0]
    d = jnp.zeros((tt, ns), jnp.float32)
    for c in range(3):
        diff = tgt_ref[:, c:c + 1] - srcT_ref[c:c + 1, :]
        d = d + diff * diff
    big = jnp.float32(1e30)
    mcur = d
    for _ in range(k - 1):
        m = jnp.min(mcur, axis=1, keepdims=True)
        mcur = jnp.where(mcur == m, big, mcur)
    m3 = jnp.min(mcur, axis=1, keepdims=True)
    wmat = jnp.where(d <= m3, 1.0 / (d + 1e-8), 0.0)
    wsum = jnp.sum(wmat, axis=1, keepdims=True)
    a = _dot(wmat, feat_ref[...], lax.Precision.DEFAULT)
    a = a * (1.0 / wsum)
    return _dot(a, wp_ref[...], lax.Precision.DEFAULT) + bp_ref[...]


def _front_body(tgt_ref,
                s0_ref, f0_ref, wp0_ref, bp0_ref,
                s1_ref, f1_ref, wp1_ref, bp1_ref,
                s2_ref, f2_ref, wp2_ref, bp2_ref,
                w1_ref, b1_ref, w2_ref, b2_ref,
                out_ref, *, ns0, ns1, ns2, c):
    a0 = _interp_block(tgt_ref, s0_ref, f0_ref, wp0_ref, bp0_ref, ns=ns0)
    a1 = _interp_block(tgt_ref, s1_ref, f1_ref, wp1_ref, bp1_ref, ns=ns1)
    a2 = _interp_block(tgt_ref, s2_ref, f2_ref, wp2_ref, bp2_ref, ns=ns2)
    h = (_dot(a0, w1_ref[0:c, :], lax.Precision.DEFAULT)
         + _dot(a1, w1_ref[c:2 * c, :], lax.Precision.DEFAULT)
         + _dot(a2, w1_ref[2 * c:3 * c, :], lax.Precision.DEFAULT)
         + b1_ref[...])
    h = 0.5 * h * (1.0 + lax.erf(h * jnp.float32(1.0 / math.sqrt(2.0))))
    out_ref[...] = _dot(h, w2_ref[...], lax.Precision.DEFAULT) + b2_ref[...]


def _front(tgt, srcs, feats, wps, bps, wf1, bf1_2d, wf2, bf2_2d, tt):
    nt = tgt.shape[0]
    c = wps[0].shape[1]
    ns = [f.shape[0] for f in feats]
    cs = [f.shape[1] for f in feats]
    full = lambda shape: pl.BlockSpec(shape, lambda i: (0, 0))
    in_specs = [pl.BlockSpec((tt, 3), lambda i: (i, 0))]
    operands = [tgt]
    for l in range(3):
        in_specs += [full((3, ns[l])), full((ns[l], cs[l])),
                     full((cs[l], c)), full((1, c))]
        operands += [srcs[l], feats[l], wps[l], bps[l]]
    in_specs += [full((3 * c, c)), full((1, c)), full((c, c)), full((1, c))]
    operands += [wf1, bf1_2d, wf2, bf2_2d]
    return pl.pallas_call(
        functools.partial(_front_body, ns0=ns[0], ns1=ns[1], ns2=ns[2], c=c),
        grid=(nt // tt,),
        in_specs=in_specs,
        out_specs=pl.BlockSpec((tt, c), lambda i: (i, 0)),
        out_shape=jax.ShapeDtypeStruct((nt, c), jnp.float32),
    )(*operands)


import numpy as np


def _trig_consts(n, n1, n2):
    i1 = np.arange(n1)
    ph1 = 2.0 * np.pi * np.outer(i1, i1) / n1
    i2 = np.arange(n2)
    ph2 = 2.0 * np.pi * np.outer(i2, i2) / n2
    p = np.arange(n)
    php = 2.0 * np.pi * ((p // n2) * (p % n2) % n) / n
    f32 = lambda a: jnp.asarray(a, dtype=jnp.float32)
    return (f32(np.cos(ph1)), f32(np.sin(ph1)),
            f32(np.cos(ph2)), f32(np.sin(ph2)),
            f32(np.cos(php)[:, None]), f32(np.sin(php)[:, None]))


def _s1_body(c1_ref, s1_ref, x_ref, ar_ref, ai_ref):
    x = x_ref[...]
    ar_ref[...] = _dot(c1_ref[...], x)
    ai_ref[...] = -_dot(s1_ref[...], x)


def _s1(c1, s1, x2, ch):
    n1, m = x2.shape
    outs = (jax.ShapeDtypeStruct((n1, m), jnp.float32),) * 2
    return pl.pallas_call(
        _s1_body,
        grid=(m // ch,),
        in_specs=[
            pl.BlockSpec((n1, n1), lambda j: (0, 0)),
            pl.BlockSpec((n1, n1), lambda j: (0, 0)),
            pl.BlockSpec((n1, ch), lambda j: (0, j)),
        ],
        out_specs=[pl.BlockSpec((n1, ch), lambda j: (0, j))] * 2,
        out_shape=outs,
    )(c1, s1, x2)


def _s23_body(c2_ref, s2_ref, ar_ref, ai_ref, tc_ref, ts_ref,
              w1_ref, b1_ref, w2_ref, b2_ref, hr_ref, hi_ref):
    ar, ai = ar_ref[...], ai_ref[...]
    tc, ts = tc_ref[...], ts_ref[...]
    br = tc * ar + ts * ai
    bi = tc * ai - ts * ar
    xr = _dot(c2_ref[...], br) + _dot(s2_ref[...], bi)
    h = jnp.maximum(_dot(xr, w1_ref[...], lax.Precision.DEFAULT)
                    + b1_ref[...], 0.0)
    e = _dot(h, w2_ref[...], lax.Precision.DEFAULT) + b2_ref[...]
    gr = _dot(c2_ref[...], e)
    gi = -_dot(s2_ref[...], e)
    hr_ref[...] = tc * gr + ts * gi
    hi_ref[...] = tc * gi - ts * gr


def _s23(c2, s2, ar, ai, tc, ts, wq1, bq1_2d, wq2, bq2_2d, n1, n2):
    n, c = ar.shape
    outs = (jax.ShapeDtypeStruct((n, c), jnp.float32),) * 2
    return pl.pallas_call(
        _s23_body,
        grid=(n1,),
        in_specs=[
            pl.BlockSpec((n2, n2), lambda k: (0, 0)),
            pl.BlockSpec((n2, n2), lambda k: (0, 0)),
            pl.BlockSpec((n2, c), lambda k: (k, 0)),
            pl.BlockSpec((n2, c), lambda k: (k, 0)),
            pl.BlockSpec((n2, 1), lambda k: (k, 0)),
            pl.BlockSpec((n2, 1), lambda k: (k, 0)),
            pl.BlockSpec((c, c), lambda k: (0, 0)),
            pl.BlockSpec((1, c), lambda k: (0, 0)),
            pl.BlockSpec((c, c), lambda k: (0, 0)),
            pl.BlockSpec((1, c), lambda k: (0, 0)),
        ],
        out_specs=[pl.BlockSpec((n2, c), lambda k: (k, 0))] * 2,
        out_shape=outs,
    )(c2, s2, ar, ai, tc, ts, wq1, bq1_2d, wq2, bq2_2d)


def _s4_body(c1_ref, s1_ref, hr_ref, hi_ref, f_ref, out_ref, *, n):
    r = (_dot(c1_ref[...], hr_ref[...]) + _dot(s1_ref[...], hi_ref[...]))
    r = r * jnp.float32(1.0 / n)
    out_ref[...] = f_ref[...] * (1.0 + jax.nn.sigmoid(r))


def _s4(c1, s1, hr2, hi2, fused2, ch, n):
    n1, m = hr2.shape
    return pl.pallas_call(
        functools.partial(_s4_body, n=n),
        grid=(m // ch,),
        in_specs=[
            pl.BlockSpec((n1, n1), lambda j: (0, 0)),
            pl.BlockSpec((n1, n1), lambda j: (0, 0)),
            pl.BlockSpec((n1, ch), lambda j: (0, j)),
            pl.BlockSpec((n1, ch), lambda j: (0, j)),
            pl.BlockSpec((n1, ch), lambda j: (0, j)),
        ],
        out_specs=pl.BlockSpec((n1, ch), lambda j: (0, j)),
        out_shape=jax.ShapeDtypeStruct((n1, m), jnp.float32),
    )(c1, s1, hr2, hi2, fused2)


def kernel(coord0, feat0, coord1, feat1, coord2, feat2, target_coord,
           Wp0, bp0, Wp1, bp1, Wp2, bp2, Wf1, bf1, Wf2, bf2,
           Wq1, bq1, Wq2, bq2):
    nt = target_coord.shape[0]
    tt_i = min(256, nt)
    tt_d = min(256, nt)

    fused = _front(target_coord,
                   (coord0.T, coord1.T, coord2.T),
                   (feat0, feat1, feat2),
                   (Wp0, Wp1, Wp2),
                   (bp0.reshape(1, -1), bp1.reshape(1, -1), bp2.reshape(1, -1)),
                   Wf1, bf1.reshape(1, -1), Wf2, bf2.reshape(1, -1), tt_i)

    c = fused.shape[1]
    if nt % (64 * 128) == 0:
        n1, n2 = 64, nt // 64
    else:
        n1 = max(d for d in range(1, int(math.isqrt(nt)) + 1) if nt % d == 0)
        n2 = nt // n1
    c1, s1, c2, s2, tw_c, tw_s = _trig_consts(nt, n1, n2)
    ch = min(2048, n2 * c)

    x2 = fused.reshape(n1, n2 * c)
    ar2, ai2 = _s1(c1, s1, x2, ch)
    hr, hi = _s23(c2, s2, ar2.reshape(nt, c), ai2.reshape(nt, c), tw_c, tw_s,
                  Wq1, bq1.reshape(1, -1), Wq2, bq2.reshape(1, -1), n1, n2)
    out2 = _s4(c1, s1, hr.reshape(n1, n2 * c), hi.reshape(n1, n2 * c),
               x2, ch, nt)
    return out2.reshape(nt, c)


# speedup vs baseline: 12.2329x; 1.0422x over previous
import functools
import math

import jax
import jax.numpy as jnp
from jax import lax
from jax.experimental import pallas as pl

_HI = lax.Precision.DEFAULT


def _dot(a, b, prec=_HI):
    return jnp.dot(a, b, precision=prec, preferred_element_type=jnp.float32)


def _interp_block(tgt_ref, srcT_ref, feat_ref, wp_ref, bp_ref, *, ns, k=3):
    tt = tgt_ref.shape[0]
    d = jnp.zeros((tt, ns), jnp.float32)
    for c in range(3):
        diff = tgt_ref[:, c:c + 1] - srcT_ref[c:c + 1, :]
        d = d + diff * diff
    big = jnp.float32(1e30)
    mcur = d
    for _ in range(k - 1):
        m = jnp.min(mcur, axis=1, keepdims=True)
        mcur = jnp.where(mcur == m, big, mcur)
    m3 = jnp.min(mcur, axis=1, keepdims=True)
    wmat = jnp.where(d <= m3, 1.0 / (d + 1e-8), 0.0)
    wsum = jnp.sum(wmat, axis=1, keepdims=True)
    a = _dot(wmat, feat_ref[...], lax.Precision.DEFAULT)
    a = a * (1.0 / wsum)
    return _dot(a, wp_ref[...], lax.Precision.DEFAULT) + bp_ref[...]


def _front_body(tgt_ref,
                s0_ref, f0_ref, wp0_ref, bp0_ref,
                s1_ref, f1_ref, wp1_ref, bp1_ref,
                s2_ref, f2_ref, wp2_ref, bp2_ref,
                w1_ref, b1_ref, w2_ref, b2_ref,
                out_ref, *, ns0, ns1, ns2, c):
    a0 = _interp_block(tgt_ref, s0_ref, f0_ref, wp0_ref, bp0_ref, ns=ns0)
    a1 = _interp_block(tgt_ref, s1_ref, f1_ref, wp1_ref, bp1_ref, ns=ns1)
    a2 = _interp_block(tgt_ref, s2_ref, f2_ref, wp2_ref, bp2_ref, ns=ns2)
    h = (_dot(a0, w1_ref[0:c, :], lax.Precision.DEFAULT)
         + _dot(a1, w1_ref[c:2 * c, :], lax.Precision.DEFAULT)
         + _dot(a2, w1_ref[2 * c:3 * c, :], lax.Precision.DEFAULT)
         + b1_ref[...])
    h = 0.5 * h * (1.0 + lax.erf(h * jnp.float32(1.0 / math.sqrt(2.0))))
    out_ref[...] = _dot(h, w2_ref[...], lax.Precision.DEFAULT) + b2_ref[...]


def _front(tgt, srcs, feats, wps, bps, wf1, bf1_2d, wf2, bf2_2d, tt):
    nt = tgt.shape[0]
    c = wps[0].shape[1]
    ns = [f.shape[0] for f in feats]
    cs = [f.shape[1] for f in feats]
    full = lambda shape: pl.BlockSpec(shape, lambda i: (0, 0))
    in_specs = [pl.BlockSpec((tt, 3), lambda i: (i, 0))]
    operands = [tgt]
    for l in range(3):
        in_specs += [full((3, ns[l])), full((ns[l], cs[l])),
                     full((cs[l], c)), full((1, c))]
        operands += [srcs[l], feats[l], wps[l], bps[l]]
    in_specs += [full((3 * c, c)), full((1, c)), full((c, c)), full((1, c))]
    operands += [wf1, bf1_2d, wf2, bf2_2d]
    return pl.pallas_call(
        functools.partial(_front_body, ns0=ns[0], ns1=ns[1], ns2=ns[2], c=c),
        grid=(nt // tt,),
        in_specs=in_specs,
        out_specs=pl.BlockSpec((tt, c), lambda i: (i, 0)),
        out_shape=jax.ShapeDtypeStruct((nt, c), jnp.float32),
    )(*operands)


import numpy as np


def _trig_consts(n, n1, n2):
    i1 = np.arange(n1)
    ph1 = 2.0 * np.pi * np.outer(i1, i1) / n1
    i2 = np.arange(n2)
    ph2 = 2.0 * np.pi * np.outer(i2, i2) / n2
    p = np.arange(n)
    php = 2.0 * np.pi * ((p // n2) * (p % n2) % n) / n
    f32 = lambda a: jnp.asarray(a, dtype=jnp.float32)
    return (f32(np.cos(ph1)), f32(np.sin(ph1)),
            f32(np.cos(ph2)), f32(np.sin(ph2)),
            f32(np.cos(php)[:, None]), f32(np.sin(php)[:, None]))


def _s1_body(c1_ref, s1_ref, x_ref, ar_ref, ai_ref):
    x = x_ref[...]
    ar_ref[...] = _dot(c1_ref[...], x)
    ai_ref[...] = -_dot(s1_ref[...], x)


def _s1(c1, s1, x2, ch):
    n1, m = x2.shape
    outs = (jax.ShapeDtypeStruct((n1, m), jnp.float32),) * 2
    return pl.pallas_call(
        _s1_body,
        grid=(m // ch,),
        in_specs=[
            pl.BlockSpec((n1, n1), lambda j: (0, 0)),
            pl.BlockSpec((n1, n1), lambda j: (0, 0)),
            pl.BlockSpec((n1, ch), lambda j: (0, j)),
        ],
        out_specs=[pl.BlockSpec((n1, ch), lambda j: (0, j))] * 2,
        out_shape=outs,
    )(c1, s1, x2)


def _s23_body(c2_ref, s2_ref, ar_ref, ai_ref, tc_ref, ts_ref,
              w1_ref, b1_ref, w2_ref, b2_ref, hr_ref, hi_ref):
    ar, ai = ar_ref[...], ai_ref[...]
    tc, ts = tc_ref[...], ts_ref[...]
    br = tc * ar + ts * ai
    bi = tc * ai - ts * ar
    xr = _dot(c2_ref[...], br) + _dot(s2_ref[...], bi)
    h = jnp.maximum(_dot(xr, w1_ref[...], lax.Precision.DEFAULT)
                    + b1_ref[...], 0.0)
    e = _dot(h, w2_ref[...], lax.Precision.DEFAULT) + b2_ref[...]
    gr = _dot(c2_ref[...], e)
    gi = -_dot(s2_ref[...], e)
    hr_ref[...] = tc * gr + ts * gi
    hi_ref[...] = tc * gi - ts * gr


def _s23(c2, s2, ar, ai, tc, ts, wq1, bq1_2d, wq2, bq2_2d, n1, n2):
    n, c = ar.shape
    outs = (jax.ShapeDtypeStruct((n, c), jnp.float32),) * 2
    return pl.pallas_call(
        _s23_body,
        grid=(n1,),
        in_specs=[
            pl.BlockSpec((n2, n2), lambda k: (0, 0)),
            pl.BlockSpec((n2, n2), lambda k: (0, 0)),
            pl.BlockSpec((n2, c), lambda k: (k, 0)),
            pl.BlockSpec((n2, c), lambda k: (k, 0)),
            pl.BlockSpec((n2, 1), lambda k: (k, 0)),
            pl.BlockSpec((n2, 1), lambda k: (k, 0)),
            pl.BlockSpec((c, c), lambda k: (0, 0)),
            pl.BlockSpec((1, c), lambda k: (0, 0)),
            pl.BlockSpec((c, c), lambda k: (0, 0)),
            pl.BlockSpec((1, c), lambda k: (0, 0)),
        ],
        out_specs=[pl.BlockSpec((n2, c), lambda k: (k, 0))] * 2,
        out_shape=outs,
    )(c2, s2, ar, ai, tc, ts, wq1, bq1_2d, wq2, bq2_2d)


def _s4_body(c1_ref, s1_ref, hr_ref, hi_ref, f_ref, out_ref, *, n):
    r = (_dot(c1_ref[...], hr_ref[...]) + _dot(s1_ref[...], hi_ref[...]))
    r = r * jnp.float32(1.0 / n)
    out_ref[...] = f_ref[...] * (1.0 + jax.nn.sigmoid(r))


def _s4(c1, s1, hr2, hi2, fused2, ch, n):
    n1, m = hr2.shape
    return pl.pallas_call(
        functools.partial(_s4_body, n=n),
        grid=(m // ch,),
        in_specs=[
            pl.BlockSpec((n1, n1), lambda j: (0, 0)),
            pl.BlockSpec((n1, n1), lambda j: (0, 0)),
            pl.BlockSpec((n1, ch), lambda j: (0, j)),
            pl.BlockSpec((n1, ch), lambda j: (0, j)),
            pl.BlockSpec((n1, ch), lambda j: (0, j)),
        ],
        out_specs=pl.BlockSpec((n1, ch), lambda j: (0, j)),
        out_shape=jax.ShapeDtypeStruct((n1, m), jnp.float32),
    )(c1, s1, hr2, hi2, fused2)


def kernel(coord0, feat0, coord1, feat1, coord2, feat2, target_coord,
           Wp0, bp0, Wp1, bp1, Wp2, bp2, Wf1, bf1, Wf2, bf2,
           Wq1, bq1, Wq2, bq2):
    nt = target_coord.shape[0]
    tt_i = min(256, nt)
    tt_d = min(256, nt)

    fused = _front(target_coord,
                   (coord0.T, coord1.T, coord2.T),
                   (feat0, feat1, feat2),
                   (Wp0, Wp1, Wp2),
                   (bp0.reshape(1, -1), bp1.reshape(1, -1), bp2.reshape(1, -1)),
                   Wf1, bf1.reshape(1, -1), Wf2, bf2.reshape(1, -1), tt_i)

    c = fused.shape[1]
    if nt % (64 * 128) == 0:
        n1, n2 = 64, nt // 64
    else:
        n1 = max(d for d in range(1, int(math.isqrt(nt)) + 1) if nt % d == 0)
        n2 = nt // n1
    c1, s1, c2, s2, tw_c, tw_s = _trig_consts(nt, n1, n2)
    ch = min(8192, n2 * c)

    x2 = fused.reshape(n1, n2 * c)
    ar2, ai2 = _s1(c1, s1, x2, ch)
    hr, hi = _s23(c2, s2, ar2.reshape(nt, c), ai2.reshape(nt, c), tw_c, tw_s,
                  Wq1, bq1.reshape(1, -1), Wq2, bq2.reshape(1, -1), n1, n2)
    out2 = _s4(c1, s1, hr.reshape(n1, n2 * c), hi.reshape(n1, n2 * c),
               x2, ch, nt)
    return out2.reshape(nt, c)
